# baseline (speedup 1.0000x reference)
import functools

import jax
import jax.numpy as jnp
from jax import lax
from jax.experimental import pallas as pl
from jax.experimental.pallas import tpu as pltpu
from jax.experimental.pallas import tpu_sc as plsc

N = 10000
D = 128
E = 320000
G = 64

NC = 2
NS = 16
NW = NC * NS
K = 64
CPS = 16
RPW0 = 320
RPW1 = 0
EP = NS * (RPW0 + RPW1) * K
ROWS_K = EP // K
NBUF = 4
NP = 10240
RPS = NP // NS
BLK = 400
NBLK = N // BLK
BN_EPS = 1e-5


_SPLAT_DNUMS = lax.GatherDimensionNumbers(
    offset_dims=(), collapsed_slice_dims=(0,), start_index_map=(0,))


def _splat(v16, j):
    idx = jnp.full((16, 1), j, dtype=jnp.int32)
    return lax.gather(v16, idx, _SPLAT_DNUMS, (1,),
                      mode=lax.GatherScatterMode.PROMISE_IN_BOUNDS)


def _sc_aggregate(h, srcm, dstm, ewm):
    mesh = plsc.VectorSubcoreMesh(core_axis_name="c", subcore_axis_name="s")

    @functools.partial(
        pl.kernel,
        out_type=jax.ShapeDtypeStruct((NC, NP, D), jnp.float32),
        mesh=mesh,
        scratch_types=[
            pltpu.VMEM((CPS, K), jnp.int32),
            pltpu.VMEM((CPS, K), jnp.int32),
            pltpu.VMEM((CPS, K), jnp.float32),
            pltpu.VMEM((NBUF, K, D), jnp.float32),
            pltpu.VMEM_SHARED((NP, D), jnp.float32),
            pltpu.SemaphoreType.DMA,
            pltpu.SemaphoreType.DMA,
        ],
    )
    def k(h_hbm, src_hbm, dst_hbm, ew_hbm, out_hbm,
          idxs, idxd, eww, rows, acc, semg, sems):
        ci = lax.axis_index("c")
        si = lax.axis_index("s")
        row0 = jnp.where(ci == 0, si * RPW0, NS * RPW0 + si * RPW1)
        nsup = jnp.where(ci == 0, RPW0 // CPS, RPW1 // CPS)

        zv = jnp.zeros((16,), jnp.float32)

        def zrow(r, carry):
            for cc in range(D // 16):
                rows[0, r, pl.ds(cc * 16, 16)] = zv
            return carry

        lax.fori_loop(0, K, zrow, 0)

        def zcopy(zz, carry):
            pltpu.sync_copy(rows.at[0],
                            acc.at[pl.ds(si * RPS + zz * K, K)])
            return carry

        lax.fori_loop(0, RPS // K, zcopy, 0)
        plsc.subcore_barrier()

        def scale(q, c):
            def grp(g, inner):
                e16 = eww[c, pl.ds(g * 16, 16)]
                for j in range(16):
                    s = _splat(e16, j)
                    r = g * 16 + j
                    for cc in range(D // 16):
                        rows[q, r, pl.ds(cc * 16, 16)] = (
                            rows[q, r, pl.ds(cc * 16, 16)] * s)
                return inner

            lax.fori_loop(0, K // 16, grp, 0)

        def gather_start(q, c):
            pltpu.async_copy(h_hbm.at[idxs.at[c]], rows.at[q], semg)

        def gather_wait(q, c):
            pltpu.make_async_copy(h_hbm.at[idxs.at[c]], rows.at[q],
                                  semg).wait()

        def scatter_start(q, c):
            pltpu.async_copy(rows.at[q], acc.at[idxd.at[c]], sems, add=True)

        def scatter_wait(q, c):
            pltpu.make_async_copy(rows.at[q], acc.at[idxd.at[c]],
                                  sems).wait()

        def superchunk(sb, carry):
            rb = row0 + sb * CPS
            pltpu.sync_copy(src_hbm.at[pl.ds(rb, CPS)], idxs)
            pltpu.sync_copy(dst_hbm.at[pl.ds(rb, CPS)], idxd)
            pltpu.sync_copy(ew_hbm.at[pl.ds(rb, CPS)], eww)
            for q in range(NBUF):
                gather_start(q, q)

            def quad(t, inner):
                c0 = t * NBUF
                for q in range(NBUF):
                    gather_wait(q, c0 + q)
                    scale(q, c0 + q)
                    scatter_start(q, c0 + q)
                for q in range(NBUF):
                    scatter_wait(q, c0 + q)
                    gather_start(q, c0 + NBUF + q)
                return inner

            lax.fori_loop(0, CPS // NBUF - 1, quad, 0)
            c0 = CPS - NBUF
            for q in range(NBUF):
                gather_wait(q, c0 + q)
                scale(q, c0 + q)
                scatter_start(q, c0 + q)
            for q in range(NBUF):
                scatter_wait(q, c0 + q)
            return carry

        lax.fori_loop(0, nsup, superchunk, 0)

        plsc.subcore_barrier()
        pltpu.sync_copy(acc.at[pl.ds(si * RPS, RPS)],
                        out_hbm.at[ci, pl.ds(si * RPS, RPS)])

    return k(h, srcm, dstm, ewm)


def _mlp_stats(h, a0, a1, W1, b1, W2, b2):
    def body(h_ref, a0_ref, a1_ref, w1_ref, b1_ref, w2_ref, b2_ref,
             z_ref, st_ref):
        i = pl.program_id(0)
        zin = h_ref[...] + a0_ref[...] + a1_ref[...]
        t = jnp.dot(zin, w1_ref[...], preferred_element_type=jnp.float32)
        t = jnp.maximum(t + b1_ref[...], 0.0)
        u = jnp.dot(t, w2_ref[...], preferred_element_type=jnp.float32)
        u = jnp.maximum(u + b2_ref[...], 0.0)
        z_ref[...] = u

        @pl.when(i == 0)
        def _():
            st_ref[...] = jnp.zeros_like(st_ref)

        su = jnp.sum(u, axis=0, keepdims=True)
        sq = jnp.sum(u * u, axis=0, keepdims=True)
        st_ref[...] += jnp.concatenate([su, sq], axis=0)

    return pl.pallas_call(
        body,
        grid=(NBLK,),
        in_specs=[
            pl.BlockSpec((BLK, D), lambda i: (i, 0)),
            pl.BlockSpec((BLK, D), lambda i: (i, 0)),
            pl.BlockSpec((BLK, D), lambda i: (i, 0)),
            pl.BlockSpec((D, D), lambda i: (0, 0)),
            pl.BlockSpec((1, D), lambda i: (0, 0)),
            pl.BlockSpec((D, D), lambda i: (0, 0)),
            pl.BlockSpec((1, D), lambda i: (0, 0)),
        ],
        out_specs=[
            pl.BlockSpec((BLK, D), lambda i: (i, 0)),
            pl.BlockSpec((2, D), lambda i: (0, 0)),
        ],
        out_shape=[
            jax.ShapeDtypeStruct((N, D), jnp.float32),
            jax.ShapeDtypeStruct((2, D), jnp.float32),
        ],
    )(h, a0, a1, W1, b1, W2, b2)


def _bn_pool(z, st, gamma, beta, batch2d):
    def body(z_ref, st_ref, g_ref, b_ref, bt_ref, h_ref, p_ref):
        i = pl.program_id(0)
        mean = st_ref[pl.ds(0, 1), :] * (1.0 / N)
        var = st_ref[pl.ds(1, 1), :] * (1.0 / N) - mean * mean
        a = g_ref[...] * lax.rsqrt(var + BN_EPS)
        c = b_ref[...] - mean * a
        hh = z_ref[...] * a + c
        h_ref[...] = hh
        seg = bt_ref[0]
        oh = (lax.broadcasted_iota(jnp.int32, (G, BLK), 0) == seg)
        pm = jnp.dot(oh.astype(jnp.float32), hh,
                     preferred_element_type=jnp.float32,
                     precision=lax.Precision.HIGHEST)

        @pl.when(i == 0)
        def _():
            p_ref[...] = jnp.zeros_like(p_ref)

        p_ref[...] += pm

    return pl.pallas_call(
        body,
        grid=(NBLK,),
        in_specs=[
            pl.BlockSpec((BLK, D), lambda i: (i, 0)),
            pl.BlockSpec((2, D), lambda i: (0, 0)),
            pl.BlockSpec((1, D), lambda i: (0, 0)),
            pl.BlockSpec((1, D), lambda i: (0, 0)),
            pl.BlockSpec((1, 1, BLK), lambda i: (i, 0, 0)),
        ],
        out_specs=[
            pl.BlockSpec((BLK, D), lambda i: (i, 0)),
            pl.BlockSpec((G, D), lambda i: (0, 0)),
        ],
        out_shape=[
            jax.ShapeDtypeStruct((N, D), jnp.float32),
            jax.ShapeDtypeStruct((G, D), jnp.float32),
        ],
    )(z, st, gamma, beta, batch2d)


def kernel(x, adj_t, batch, edge_weight,
           W1_0, b1_0, W2_0, b2_0, gamma_0, beta_0,
           W1_1, b1_1, W2_1, b2_1, gamma_1, beta_1):
    pad = EP - E
    srcm = jnp.concatenate(
        [adj_t[0], jnp.zeros((pad,), jnp.int32)]).reshape(ROWS_K, K)
    dstm = jnp.concatenate(
        [adj_t[1], jnp.zeros((pad,), jnp.int32)]).reshape(ROWS_K, K)
    ewm = jnp.concatenate(
        [edge_weight, jnp.zeros((pad,), jnp.float32)]).reshape(ROWS_K, K)
    bt = batch.reshape(NBLK, 1, BLK)

    params = [
        (W1_0, b1_0, W2_0, b2_0, gamma_0, beta_0),
        (W1_1, b1_1, W2_1, b2_1, gamma_1, beta_1),
    ]
    h = x
    pools = []
    for (W1, b1, W2, b2, g, b) in params:
        agg = _sc_aggregate(h, srcm, dstm, ewm)
        z, st = _mlp_stats(h, agg[0, :N], agg[1, :N], W1, b1.reshape(1, D),
                           W2, b2.reshape(1, D))
        h, pool = _bn_pool(z, st, g.reshape(1, D), b.reshape(1, D), bt)
        pools.append(pool)
    return h, jnp.concatenate(pools, axis=1)


# speedup vs baseline: 3.1103x; 3.1103x over previous
import functools

import jax
import jax.numpy as jnp
from jax import lax
from jax.experimental import pallas as pl
from jax.experimental.pallas import tpu as pltpu
from jax.experimental.pallas import tpu_sc as plsc

N = 10000
D = 128
E = 320000
G = 64

NC = 2
NS = 16
NW = NC * NS
K = 64
CPS = 16
RPW0 = 160
RPW1 = 160
EP = NS * (RPW0 + RPW1) * K
ROWS_K = EP // K
NBUF = 4
NP = 10240
RPS = NP // NS
BLK = 400
NBLK = N // BLK
BN_EPS = 1e-5


_SPLAT_DNUMS = lax.GatherDimensionNumbers(
    offset_dims=(), collapsed_slice_dims=(0,), start_index_map=(0,))


def _splat(v16, j):
    idx = jnp.full((16, 1), j, dtype=jnp.int32)
    return lax.gather(v16, idx, _SPLAT_DNUMS, (1,),
                      mode=lax.GatherScatterMode.PROMISE_IN_BOUNDS)


def _sc_aggregate(h, srcm, dstm, ewm):
    mesh = plsc.VectorSubcoreMesh(core_axis_name="c", subcore_axis_name="s")

    @functools.partial(
        pl.kernel,
        out_type=jax.ShapeDtypeStruct((NC, NP, D), jnp.float32),
        mesh=mesh,
        scratch_types=[
            pltpu.VMEM((CPS, K), jnp.int32),
            pltpu.VMEM((CPS, K), jnp.int32),
            pltpu.VMEM((CPS, K), jnp.float32),
            pltpu.VMEM((NBUF, K, D), jnp.float32),
            pltpu.VMEM_SHARED((NP, D), jnp.float32),
            pltpu.SemaphoreType.DMA,
            pltpu.SemaphoreType.DMA,
        ],
    )
    def k(h_hbm, src_hbm, dst_hbm, ew_hbm, out_hbm,
          idxs, idxd, eww, rows, acc, semg, sems):
        ci = lax.axis_index("c")
        si = lax.axis_index("s")
        row0 = jnp.where(ci == 0, si * RPW0, NS * RPW0 + si * RPW1)
        nsup = jnp.where(ci == 0, RPW0 // CPS, RPW1 // CPS)

        zv = jnp.zeros((16,), jnp.float32)

        def zrow(r, carry):
            for cc in range(D // 16):
                rows[0, r, pl.ds(cc * 16, 16)] = zv
            return carry

        lax.fori_loop(0, K, zrow, 0)

        def zcopy(zz, carry):
            pltpu.sync_copy(rows.at[0],
                            acc.at[pl.ds(si * RPS + zz * K, K)])
            return carry

        lax.fori_loop(0, RPS // K, zcopy, 0)
        plsc.subcore_barrier()

        def scale(q, c):
            def grp(g, inner):
                e16 = eww[c, pl.ds(g * 16, 16)]
                for j in range(16):
                    s = _splat(e16, j)
                    r = g * 16 + j
                    for cc in range(D // 16):
                        rows[q, r, pl.ds(cc * 16, 16)] = (
                            rows[q, r, pl.ds(cc * 16, 16)] * s)
                return inner

            lax.fori_loop(0, K // 16, grp, 0)

        def gather_start(q, c):
            pltpu.async_copy(h_hbm.at[idxs.at[c]], rows.at[q], semg)

        def gather_wait(q, c):
            pltpu.make_async_copy(h_hbm.at[idxs.at[c]], rows.at[q],
                                  semg).wait()

        def scatter_start(q, c):
            pltpu.async_copy(rows.at[q], acc.at[idxd.at[c]], sems, add=True)

        def scatter_wait(q, c):
            pltpu.make_async_copy(rows.at[q], acc.at[idxd.at[c]],
                                  sems).wait()

        def superchunk(sb, carry):
            rb = row0 + sb * CPS
            pltpu.sync_copy(src_hbm.at[pl.ds(rb, CPS)], idxs)
            pltpu.sync_copy(dst_hbm.at[pl.ds(rb, CPS)], idxd)
            pltpu.sync_copy(ew_hbm.at[pl.ds(rb, CPS)], eww)
            for q in range(NBUF):
                gather_start(q, q)

            def quad(t, inner):
                c0 = t * NBUF
                for q in range(NBUF):
                    gather_wait(q, c0 + q)
                    scale(q, c0 + q)
                    scatter_start(q, c0 + q)
                for q in range(NBUF):
                    scatter_wait(q, c0 + q)
                    gather_start(q, c0 + NBUF + q)
                return inner

            lax.fori_loop(0, CPS // NBUF - 1, quad, 0)
            c0 = CPS - NBUF
            for q in range(NBUF):
                gather_wait(q, c0 + q)
                scale(q, c0 + q)
                scatter_start(q, c0 + q)
            for q in range(NBUF):
                scatter_wait(q, c0 + q)
            return carry

        lax.fori_loop(0, nsup, superchunk, 0)

        plsc.subcore_barrier()
        pltpu.sync_copy(acc.at[pl.ds(si * RPS, RPS)],
                        out_hbm.at[ci, pl.ds(si * RPS, RPS)])

    return k(h, srcm, dstm, ewm)


def _mlp_stats(h, a0, a1, W1, b1, W2, b2):
    def body(h_ref, a0_ref, a1_ref, w1_ref, b1_ref, w2_ref, b2_ref,
             z_ref, st_ref):
        i = pl.program_id(0)
        zin = h_ref[...] + a0_ref[...] + a1_ref[...]
        t = jnp.dot(zin, w1_ref[...], preferred_element_type=jnp.float32)
        t = jnp.maximum(t + b1_ref[...], 0.0)
        u = jnp.dot(t, w2_ref[...], preferred_element_type=jnp.float32)
        u = jnp.maximum(u + b2_ref[...], 0.0)
        z_ref[...] = u

        @pl.when(i == 0)
        def _():
            st_ref[...] = jnp.zeros_like(st_ref)

        su = jnp.sum(u, axis=0, keepdims=True)
        sq = jnp.sum(u * u, axis=0, keepdims=True)
        st_ref[...] += jnp.concatenate([su, sq], axis=0)

    return pl.pallas_call(
        body,
        grid=(NBLK,),
        in_specs=[
            pl.BlockSpec((BLK, D), lambda i: (i, 0)),
            pl.BlockSpec((BLK, D), lambda i: (i, 0)),
            pl.BlockSpec((BLK, D), lambda i: (i, 0)),
            pl.BlockSpec((D, D), lambda i: (0, 0)),
            pl.BlockSpec((1, D), lambda i: (0, 0)),
            pl.BlockSpec((D, D), lambda i: (0, 0)),
            pl.BlockSpec((1, D), lambda i: (0, 0)),
        ],
        out_specs=[
            pl.BlockSpec((BLK, D), lambda i: (i, 0)),
            pl.BlockSpec((2, D), lambda i: (0, 0)),
        ],
        out_shape=[
            jax.ShapeDtypeStruct((N, D), jnp.float32),
            jax.ShapeDtypeStruct((2, D), jnp.float32),
        ],
    )(h, a0, a1, W1, b1, W2, b2)


def _bn_pool(z, st, gamma, beta, batch2d):
    def body(z_ref, st_ref, g_ref, b_ref, bt_ref, h_ref, p_ref):
        i = pl.program_id(0)
        mean = st_ref[pl.ds(0, 1), :] * (1.0 / N)
        var = st_ref[pl.ds(1, 1), :] * (1.0 / N) - mean * mean
        a = g_ref[...] * lax.rsqrt(var + BN_EPS)
        c = b_ref[...] - mean * a
        hh = z_ref[...] * a + c
        h_ref[...] = hh
        seg = bt_ref[0]
        oh = (lax.broadcasted_iota(jnp.int32, (G, BLK), 0) == seg)
        pm = jnp.dot(oh.astype(jnp.float32), hh,
                     preferred_element_type=jnp.float32,
                     precision=lax.Precision.HIGHEST)

        @pl.when(i == 0)
        def _():
            p_ref[...] = jnp.zeros_like(p_ref)

        p_ref[...] += pm

    return pl.pallas_call(
        body,
        grid=(NBLK,),
        in_specs=[
            pl.BlockSpec((BLK, D), lambda i: (i, 0)),
            pl.BlockSpec((2, D), lambda i: (0, 0)),
            pl.BlockSpec((1, D), lambda i: (0, 0)),
            pl.BlockSpec((1, D), lambda i: (0, 0)),
            pl.BlockSpec((1, 1, BLK), lambda i: (i, 0, 0)),
        ],
        out_specs=[
            pl.BlockSpec((BLK, D), lambda i: (i, 0)),
            pl.BlockSpec((G, D), lambda i: (0, 0)),
        ],
        out_shape=[
            jax.ShapeDtypeStruct((N, D), jnp.float32),
            jax.ShapeDtypeStruct((G, D), jnp.float32),
        ],
    )(z, st, gamma, beta, batch2d)


def kernel(x, adj_t, batch, edge_weight,
           W1_0, b1_0, W2_0, b2_0, gamma_0, beta_0,
           W1_1, b1_1, W2_1, b2_1, gamma_1, beta_1):
    pad = EP - E
    srcm = jnp.concatenate(
        [adj_t[0],
         jnp.arange(pad, dtype=jnp.int32) % N]).reshape(ROWS_K, K)
    dstm = jnp.concatenate(
        [adj_t[1],
         N + (jnp.arange(pad, dtype=jnp.int32) % (NP - N))]).reshape(
             ROWS_K, K)
    ewm = jnp.concatenate(
        [edge_weight, jnp.zeros((pad,), jnp.float32)]).reshape(ROWS_K, K)
    bt = batch.reshape(NBLK, 1, BLK)

    params = [
        (W1_0, b1_0, W2_0, b2_0, gamma_0, beta_0),
        (W1_1, b1_1, W2_1, b2_1, gamma_1, beta_1),
    ]
    h = x
    pools = []
    for (W1, b1, W2, b2, g, b) in params:
        agg = _sc_aggregate(h, srcm, dstm, ewm)
        z, st = _mlp_stats(h, agg[0, :N], agg[1, :N], W1, b1.reshape(1, D),
                           W2, b2.reshape(1, D))
        h, pool = _bn_pool(z, st, g.reshape(1, D), b.reshape(1, D), bt)
        pools.append(pool)
    return h, jnp.concatenate(pools, axis=1)
